# baseline (speedup 1.0000x reference)
import jax
import jax.numpy as jnp
from jax import lax
from jax.experimental import pallas as pl
from jax.experimental.pallas import tpu as pltpu

SEQ = 9
DM = 1024
SEG = 2048
BLKR = 4032
PAD = 16


def _body(x_ref, w_ref, o_ref, pe_ref):
    i = pl.program_id(0)

    @pl.when(i == 0)
    def _init():
        w = w_ref[:, 0, :]
        ss = jnp.sum(w * w, axis=-1, keepdims=True)
        norm = jnp.sqrt(ss)
        scale = jnp.where(norm > 1.0, 1.0 / (norm + 1e-7), 1.0)
        pe_ref[...] = w * scale

    base = i * BLKR
    s0 = base // SEG
    r0 = lax.rem(s0, SEQ)
    r1 = lax.rem(s0 + 1, SEQ)
    r2 = lax.rem(s0 + 2, SEQ)
    bnd1 = SEG - lax.rem(base, SEG)
    bnd2 = bnd1 + SEG
    row0 = pe_ref[pl.ds(r0, 1), :]
    row1 = pe_ref[pl.ds(r1, 1), :]
    row2 = pe_ref[pl.ds(r2, 1), :]
    iota = lax.broadcasted_iota(jnp.int32, (BLKR, 1), 0)
    pe_blk = jnp.where(iota < bnd1, row0, jnp.where(iota < bnd2, row1, row2))
    o_ref[...] = x_ref[...] + pe_blk


def kernel(x, W):
    B = x.shape[0]
    n = B * SEQ * SEG
    xf = x.reshape(n, DM)
    Wp = jnp.zeros((PAD, 1, DM), W.dtype).at[:SEQ, 0, :].set(W)
    out = pl.pallas_call(
        _body,
        grid=((n + BLKR - 1) // BLKR,),
        in_specs=[
            pl.BlockSpec((BLKR, DM), lambda i: (i, 0)),
            pl.BlockSpec((PAD, 1, DM), lambda i: (0, 0, 0)),
        ],
        out_specs=pl.BlockSpec((BLKR, DM), lambda i: (i, 0)),
        out_shape=jax.ShapeDtypeStruct((n, DM), x.dtype),
        scratch_shapes=[pltpu.VMEM((PAD, DM), jnp.float32)],
        compiler_params=pltpu.CompilerParams(
            dimension_semantics=("arbitrary",),
            vmem_limit_bytes=67000000,
        ),
    )(xf, Wp)
    return out.reshape(x.shape)


# speedup vs baseline: 1.0036x; 1.0036x over previous
import jax
import jax.numpy as jnp
from jax import lax
from jax.experimental import pallas as pl
from jax.experimental.pallas import tpu as pltpu

SEQ = 9
DM = 1024
SEG = 2048
BLKR = 3840
PAD = 16


def _body(x_ref, w_ref, o_ref, pe_ref):
    i = pl.program_id(0)

    @pl.when(i == 0)
    def _init():
        w = w_ref[:, 0, :]
        ss = jnp.sum(w * w, axis=-1, keepdims=True)
        norm = jnp.sqrt(ss)
        scale = jnp.where(norm > 1.0, 1.0 / (norm + 1e-7), 1.0)
        pe_ref[...] = w * scale

    base = i * BLKR
    s0 = base // SEG
    r0 = lax.rem(s0, SEQ)
    r1 = lax.rem(s0 + 1, SEQ)
    r2 = lax.rem(s0 + 2, SEQ)
    bnd1 = SEG - lax.rem(base, SEG)
    bnd2 = bnd1 + SEG
    row0 = pe_ref[pl.ds(r0, 1), :]
    row1 = pe_ref[pl.ds(r1, 1), :]
    row2 = pe_ref[pl.ds(r2, 1), :]
    iota = lax.broadcasted_iota(jnp.int32, (BLKR, 1), 0)
    pe_blk = jnp.where(iota < bnd1, row0, jnp.where(iota < bnd2, row1, row2))
    o_ref[...] = x_ref[...] + pe_blk


def kernel(x, W):
    B = x.shape[0]
    n = B * SEQ * SEG
    xf = x.reshape(n, DM)
    Wp = jnp.zeros((PAD, 1, DM), W.dtype).at[:SEQ, 0, :].set(W)
    out = pl.pallas_call(
        _body,
        grid=((n + BLKR - 1) // BLKR,),
        in_specs=[
            pl.BlockSpec((BLKR, DM), lambda i: (i, 0)),
            pl.BlockSpec((PAD, 1, DM), lambda i: (0, 0, 0)),
        ],
        out_specs=pl.BlockSpec((BLKR, DM), lambda i: (i, 0)),
        out_shape=jax.ShapeDtypeStruct((n, DM), x.dtype),
        scratch_shapes=[pltpu.VMEM((PAD, DM), jnp.float32)],
        compiler_params=pltpu.CompilerParams(
            dimension_semantics=("arbitrary",),
            vmem_limit_bytes=67000000,
        ),
    )(xf, Wp)
    return out.reshape(x.shape)
